# baseline (speedup 1.0000x reference)
import functools

import jax
import jax.numpy as jnp
from jax import lax
from jax.experimental import pallas as pl
from jax.experimental.pallas import tpu as pltpu
from jax.experimental.pallas import tpu_sc as plsc

REP = 128
HID = 2 * REP
N_NODES = 10000
N_EDGES = 320000

NC = 2
NS = 16
NW = NC * NS

NSLICE = 5
E_SLICE = N_EDGES // NSLICE
EPS = E_SLICE // NW
G_CHUNK = 128
G_STEPS = [(k * G_CHUNK, G_CHUNK) for k in range(EPS // G_CHUNK)]
if EPS % G_CHUNK:
    G_STEPS.append(((EPS // G_CHUNK) * G_CHUNK, EPS % G_CHUNK))
S_CHUNK = 80
S_NCHUNK = EPS // S_CHUNK

N_NODES_PAD = 10240
NPW = N_NODES_PAD // NS


@functools.cache
def _sc_mesh():
    return plsc.VectorSubcoreMesh(core_axis_name="c", subcore_axis_name="s")


@functools.cache
def _build_sc_gather():
    @functools.partial(
        pl.kernel,
        mesh=_sc_mesh(),
        out_type=(
            jax.ShapeDtypeStruct((E_SLICE, REP), jnp.float32),
            jax.ShapeDtypeStruct((E_SLICE, REP), jnp.float32),
        ),
        scratch_types=[
            pltpu.VMEM((EPS,), jnp.int32),
            pltpu.VMEM((EPS,), jnp.int32),
            pltpu.VMEM((2, G_CHUNK, REP), jnp.float32),
            pltpu.VMEM((2, G_CHUNK, REP), jnp.float32),
            pltpu.SemaphoreType.DMA((2,)),
            pltpu.SemaphoreType.DMA((2,)),
            pltpu.SemaphoreType.DMA((2,)),
            pltpu.SemaphoreType.DMA((2,)),
        ],
    )
    def sc_gather(table, src_r, dst_r, gsrc, gdst,
                  idx_s, idx_d, rows_s, rows_d, gsem_s, gsem_d, wsem_s, wsem_d):
        c = lax.axis_index("c")
        s = lax.axis_index("s")
        base = (c * NS + s) * EPS
        pltpu.sync_copy(src_r.at[c, s], idx_s)
        pltpu.sync_copy(dst_r.at[c, s], idx_d)
        nstep = len(G_STEPS)

        def refs(k):
            off, sz = G_STEPS[k]
            p = k % 2
            return (
                (table.at[idx_s.at[pl.ds(off, sz)]],
                 rows_s.at[p, pl.ds(0, sz)], gsem_s.at[p],
                 gsrc.at[pl.ds(base + off, sz)], wsem_s.at[p]),
                (table.at[idx_d.at[pl.ds(off, sz)]],
                 rows_d.at[p, pl.ds(0, sz)], gsem_d.at[p],
                 gdst.at[pl.ds(base + off, sz)], wsem_d.at[p]),
            )

        def start_gather(k):
            for tbl, rows, gsem, _, _w in refs(k):
                pltpu.async_copy(tbl, rows, gsem)

        def wait_gather(k):
            for tbl, rows, gsem, _, _w in refs(k):
                pltpu.make_async_copy(tbl, rows, gsem).wait()

        def start_write(k):
            for _, rows, _g, dst_ref, wsem in refs(k):
                pltpu.async_copy(rows, dst_ref, wsem)

        def wait_write(k):
            for _, rows, _g, dst_ref, wsem in refs(k):
                pltpu.make_async_copy(rows, dst_ref, wsem).wait()

        start_gather(0)
        for k in range(nstep):
            if k + 1 < nstep:
                if k >= 1:
                    wait_write(k - 1)
                start_gather(k + 1)
            wait_gather(k)
            start_write(k)
        wait_write(nstep - 2)
        wait_write(nstep - 1)

    return sc_gather


@functools.cache
def _build_sc_scatter(n_sl):
    @functools.partial(
        pl.kernel,
        mesh=_sc_mesh(),
        out_type=jax.ShapeDtypeStruct((NC, N_NODES_PAD, REP), jnp.float32),
        scratch_types=[
            pltpu.VMEM((2, S_CHUNK), jnp.int32),
            pltpu.VMEM((2, S_CHUNK), jnp.int32),
            pltpu.VMEM((2, S_CHUNK, REP), jnp.float32),
            pltpu.VMEM_SHARED((N_NODES_PAD, REP), jnp.float32),
            pltpu.SemaphoreType.DMA((2,)),
            pltpu.SemaphoreType.DMA((2,)),
            pltpu.SemaphoreType.DMA((2,)),
        ],
    )
    def sc_scatter(*args):
        (args, (out, idx_s, idx_d, rows, acc, rsem, isem_s, isem_d)) = (
            args[:3 * n_sl + 1], args[3 * n_sl + 1:])
        eos = args[0:n_sl]
        srcs = args[n_sl:2 * n_sl]
        dsts = args[2 * n_sl:3 * n_sl]
        zeros = args[3 * n_sl]
        c = lax.axis_index("c")
        s = lax.axis_index("s")
        pltpu.sync_copy(zeros.at[pl.ds(s * NPW, NPW)], acc.at[pl.ds(s * NPW, NPW)])
        plsc.subcore_barrier()

        for q in range(n_sl):
            eo_r, src_r, dst_r = eos[q], srcs[q], dsts[q]

            def start_chunk(i, p, eo_r=eo_r, src_r=src_r, dst_r=dst_r):
                pltpu.async_copy(src_r.at[c, s, i], idx_s.at[p], isem_s.at[p])
                pltpu.async_copy(dst_r.at[c, s, i], idx_d.at[p], isem_d.at[p])
                pltpu.async_copy(eo_r.at[c, s, i], rows.at[p], rsem.at[p])

            def wait_chunk(i, p, eo_r=eo_r, src_r=src_r, dst_r=dst_r):
                pltpu.make_async_copy(src_r.at[c, s, i], idx_s.at[p],
                                      isem_s.at[p]).wait()
                pltpu.make_async_copy(dst_r.at[c, s, i], idx_d.at[p],
                                      isem_d.at[p]).wait()
                pltpu.make_async_copy(eo_r.at[c, s, i], rows.at[p],
                                      rsem.at[p]).wait()

            start_chunk(0, 0)

            def body(i, carry, start_chunk=start_chunk, wait_chunk=wait_chunk):
                p = i % 2

                @pl.when(i + 1 < S_NCHUNK)
                def _():
                    start_chunk(i + 1, 1 - p)

                wait_chunk(i, p)
                pltpu.sync_copy(rows.at[p], acc.at[idx_s.at[p]], add=True)
                pltpu.sync_copy(rows.at[p], acc.at[idx_d.at[p]], add=True)
                return carry

            lax.fori_loop(0, S_NCHUNK, body, 0)

        plsc.subcore_barrier()
        pltpu.sync_copy(acc.at[pl.ds(s * NPW, NPW)],
                        out.at[c].at[pl.ds(s * NPW, NPW)])

    return sc_scatter


def _edge_body(ea, gs, gd, w1a, w1b, w1c, b1, w2, b2, out):
    f32 = jnp.float32
    bf = jnp.bfloat16
    h = (jnp.dot(ea[...].astype(bf), w1a[...], preferred_element_type=f32)
         + jnp.dot(gs[...].astype(bf), w1b[...], preferred_element_type=f32)
         + jnp.dot(gd[...].astype(bf), w1c[...], preferred_element_type=f32)
         + b1[...])
    h = jnp.maximum(h, 0.0).astype(bf)
    out[...] = jnp.dot(h, w2[...], preferred_element_type=f32) + b2[...]


def _node_body(nr, p0, p1, w1a, w1b, b1, w2, b2, out):
    e2n = p0[...] + p1[...]
    g = (jnp.dot(nr[...], w1a[...], preferred_element_type=jnp.float32)
         + jnp.dot(e2n, w1b[...], preferred_element_type=jnp.float32)
         + b1[...])
    g = jnp.maximum(g, 0.0)
    out[...] = jnp.dot(g, w2[...], preferred_element_type=jnp.float32) + b2[...]


_BE = 2000
NBLK = E_SLICE // _BE
_BN = 1000


def _full(shape):
    return pl.BlockSpec(shape, lambda i: (0, 0))


def _edge_mlp_slice(q, edge_attr, gs, gd, weights):
    row_g = pl.BlockSpec((_BE, REP), lambda i, q=q: (q * NBLK + i, 0))
    row_l = pl.BlockSpec((_BE, REP), lambda i: (i, 0))
    wspecs = [_full((REP, HID)), _full((REP, HID)), _full((REP, HID)),
              _full((1, HID)), _full((HID, REP)), _full((1, REP))]
    return pl.pallas_call(
        _edge_body,
        grid=(NBLK,),
        in_specs=[row_g, row_l, row_l] + wspecs,
        out_specs=row_l,
        out_shape=jax.ShapeDtypeStruct((E_SLICE, REP), jnp.float32),
    )(edge_attr, gs, gd, *weights)


def _node_mlp(node_rep, parts, w1a, w1b, b1, w2, b2):
    row = pl.BlockSpec((_BN, REP), lambda i: (i, 0))
    return pl.pallas_call(
        _node_body,
        grid=(N_NODES // _BN,),
        in_specs=[row, row, row,
                  _full((REP, HID)), _full((REP, HID)),
                  _full((1, HID)), _full((HID, REP)), _full((1, REP))],
        out_specs=row,
        out_shape=jax.ShapeDtypeStruct((N_NODES, REP), jnp.float32),
    )(node_rep, *parts, w1a, w1b, b1, w2, b2)


def kernel(node_rep, edge_index, edge_attr, We1, be1, We2, be2, Wn1, bn1, Wn2, bn2):
    bf = jnp.bfloat16
    src = edge_index[0].astype(jnp.int32)
    dst = edge_index[1].astype(jnp.int32)
    src_g = src.reshape(NSLICE, NC, NS, EPS)
    dst_g = dst.reshape(NSLICE, NC, NS, EPS)
    src_s = src.reshape(NSLICE, NC, NS, S_NCHUNK, S_CHUNK)
    dst_s = dst.reshape(NSLICE, NC, NS, S_NCHUNK, S_CHUNK)

    sc_gather = _build_sc_gather()
    gathered = [sc_gather(node_rep, src_g[q], dst_g[q]) for q in range(NSLICE)]

    weights = (We1[:REP].astype(bf), We1[REP:2 * REP].astype(bf),
               We1[2 * REP:].astype(bf), be1.reshape(1, HID),
               We2.astype(bf), be2.reshape(1, REP))
    eo = [_edge_mlp_slice(q, edge_attr, *gathered[q], weights)
          for q in range(NSLICE)]

    zeros = jnp.zeros((N_NODES_PAD, REP), jnp.float32)
    eo_r = [e.reshape(NC, NS, S_NCHUNK, S_CHUNK, REP) for e in eo]
    partials = _build_sc_scatter(NSLICE)(
        *eo_r, *[src_s[q] for q in range(NSLICE)],
        *[dst_s[q] for q in range(NSLICE)], zeros)

    edge_out = jnp.concatenate(eo, axis=0)
    parts = (partials[0, :N_NODES], partials[1, :N_NODES])
    node_out = _node_mlp(node_rep, parts, Wn1[:REP], Wn1[REP:],
                         bn1.reshape(1, HID), Wn2, bn2.reshape(1, REP))
    return node_out, edge_out


# speedup vs baseline: 1.1020x; 1.1020x over previous
import functools

import jax
import jax.numpy as jnp
from jax import lax
from jax.experimental import pallas as pl
from jax.experimental.pallas import tpu as pltpu
from jax.experimental.pallas import tpu_sc as plsc

REP = 128
HID = 2 * REP
N_NODES = 10000
N_EDGES = 320000

NC = 2
NS = 16
NW = NC * NS

NSLICE = 5
E_SLICE = N_EDGES // NSLICE
EPS = E_SLICE // NW
G_CHUNK = 128
G_STEPS = [(k * G_CHUNK, G_CHUNK) for k in range(EPS // G_CHUNK)]
if EPS % G_CHUNK:
    G_STEPS.append(((EPS // G_CHUNK) * G_CHUNK, EPS % G_CHUNK))
S_CHUNK = 80
EPW = N_EDGES // NW
S_NCHUNK = EPW // S_CHUNK

N_NODES_PAD = 10240
NPW = N_NODES_PAD // NS


@functools.cache
def _sc_mesh():
    return plsc.VectorSubcoreMesh(core_axis_name="c", subcore_axis_name="s")


@functools.cache
def _build_sc_gather():
    @functools.partial(
        pl.kernel,
        mesh=_sc_mesh(),
        out_type=(
            jax.ShapeDtypeStruct((E_SLICE, REP), jnp.float32),
            jax.ShapeDtypeStruct((E_SLICE, REP), jnp.float32),
        ),
        scratch_types=[
            pltpu.VMEM((EPS,), jnp.int32),
            pltpu.VMEM((EPS,), jnp.int32),
            pltpu.VMEM((2, G_CHUNK, REP), jnp.float32),
            pltpu.VMEM((2, G_CHUNK, REP), jnp.float32),
            pltpu.SemaphoreType.DMA((2,)),
            pltpu.SemaphoreType.DMA((2,)),
            pltpu.SemaphoreType.DMA((2,)),
            pltpu.SemaphoreType.DMA((2,)),
        ],
    )
    def sc_gather(table, src_r, dst_r, gsrc, gdst,
                  idx_s, idx_d, rows_s, rows_d, gsem_s, gsem_d, wsem_s, wsem_d):
        c = lax.axis_index("c")
        s = lax.axis_index("s")
        base = (c * NS + s) * EPS
        pltpu.sync_copy(src_r.at[c, s], idx_s)
        pltpu.sync_copy(dst_r.at[c, s], idx_d)
        nstep = len(G_STEPS)

        def refs(k):
            off, sz = G_STEPS[k]
            p = k % 2
            return (
                (table.at[idx_s.at[pl.ds(off, sz)]],
                 rows_s.at[p, pl.ds(0, sz)], gsem_s.at[p],
                 gsrc.at[pl.ds(base + off, sz)], wsem_s.at[p]),
                (table.at[idx_d.at[pl.ds(off, sz)]],
                 rows_d.at[p, pl.ds(0, sz)], gsem_d.at[p],
                 gdst.at[pl.ds(base + off, sz)], wsem_d.at[p]),
            )

        def start_gather(k):
            for tbl, rows, gsem, _, _w in refs(k):
                pltpu.async_copy(tbl, rows, gsem)

        def wait_gather(k):
            for tbl, rows, gsem, _, _w in refs(k):
                pltpu.make_async_copy(tbl, rows, gsem).wait()

        def start_write(k):
            for _, rows, _g, dst_ref, wsem in refs(k):
                pltpu.async_copy(rows, dst_ref, wsem)

        def wait_write(k):
            for _, rows, _g, dst_ref, wsem in refs(k):
                pltpu.make_async_copy(rows, dst_ref, wsem).wait()

        start_gather(0)
        for k in range(nstep):
            if k + 1 < nstep:
                if k >= 1:
                    wait_write(k - 1)
                start_gather(k + 1)
            wait_gather(k)
            start_write(k)
        wait_write(nstep - 2)
        wait_write(nstep - 1)

    return sc_gather


@functools.cache
def _build_sc_scatter():
    @functools.partial(
        pl.kernel,
        mesh=_sc_mesh(),
        out_type=jax.ShapeDtypeStruct((NC, N_NODES_PAD, REP), jnp.float32),
        scratch_types=[
            pltpu.VMEM((2, S_CHUNK), jnp.int32),
            pltpu.VMEM((2, S_CHUNK), jnp.int32),
            pltpu.VMEM((2, S_CHUNK, REP), jnp.float32),
            pltpu.VMEM_SHARED((N_NODES_PAD, REP), jnp.float32),
            pltpu.SemaphoreType.DMA((2,)),
            pltpu.SemaphoreType.DMA((2,)),
            pltpu.SemaphoreType.DMA((2,)),
            pltpu.SemaphoreType.DMA((2,)),
            pltpu.SemaphoreType.DMA((2,)),
        ],
    )
    def sc_scatter(eo_r, src_r, dst_r, zeros, out, idx_s, idx_d, rows, acc,
                   rsem, isem_s, isem_d, asem_s, asem_d):
        c = lax.axis_index("c")
        s = lax.axis_index("s")
        pltpu.sync_copy(zeros.at[pl.ds(s * NPW, NPW)], acc.at[pl.ds(s * NPW, NPW)])
        plsc.subcore_barrier()

        def start_chunk(i, p):
            pltpu.async_copy(src_r.at[c, s, i], idx_s.at[p], isem_s.at[p])
            pltpu.async_copy(dst_r.at[c, s, i], idx_d.at[p], isem_d.at[p])
            pltpu.async_copy(eo_r.at[c, s, i], rows.at[p], rsem.at[p])

        def wait_chunk(i, p):
            pltpu.make_async_copy(src_r.at[c, s, i], idx_s.at[p],
                                  isem_s.at[p]).wait()
            pltpu.make_async_copy(dst_r.at[c, s, i], idx_d.at[p],
                                  isem_d.at[p]).wait()
            pltpu.make_async_copy(eo_r.at[c, s, i], rows.at[p],
                                  rsem.at[p]).wait()

        def start_adds(p):
            pltpu.async_copy(rows.at[p], acc.at[idx_s.at[p]], asem_s.at[p],
                             add=True)
            pltpu.async_copy(rows.at[p], acc.at[idx_d.at[p]], asem_d.at[p],
                             add=True)

        def wait_adds(p):
            pltpu.make_async_copy(rows.at[p], acc.at[idx_s.at[p]],
                                  asem_s.at[p]).wait()
            pltpu.make_async_copy(rows.at[p], acc.at[idx_d.at[p]],
                                  asem_d.at[p]).wait()

        start_chunk(0, 0)

        def body(i, carry):
            p = i % 2

            @pl.when(i + 1 < S_NCHUNK)
            def _():
                @pl.when(i >= 1)
                def _():
                    wait_adds(1 - p)
                start_chunk(i + 1, 1 - p)

            wait_chunk(i, p)
            start_adds(p)
            return carry

        lax.fori_loop(0, S_NCHUNK, body, 0)
        wait_adds(S_NCHUNK % 2)
        wait_adds((S_NCHUNK - 1) % 2)
        plsc.subcore_barrier()
        pltpu.sync_copy(acc.at[pl.ds(s * NPW, NPW)],
                        out.at[c].at[pl.ds(s * NPW, NPW)])

    return sc_scatter


def _edge_math(ea, gs, gd, w1a, w1b, w1c, b1, w2, b2, out):
    f32 = jnp.float32
    bf = jnp.bfloat16
    h = (jnp.dot(ea[...].astype(bf), w1a[...], preferred_element_type=f32)
         + jnp.dot(gs[...].astype(bf), w1b[...], preferred_element_type=f32)
         + jnp.dot(gd[...].astype(bf), w1c[...], preferred_element_type=f32)
         + b1[...])
    h = jnp.maximum(h, 0.0).astype(bf)
    out[...] = jnp.dot(h, w2[...], preferred_element_type=f32) + b2[...]


def _edge_body_first(ea, gs, gd, w1a, w1b, w1c, b1, w2, b2, out):
    _edge_math(ea, gs, gd, w1a, w1b, w1c, b1, w2, b2, out)


def _edge_body_acc(prev, ea, gs, gd, w1a, w1b, w1c, b1, w2, b2, out):
    del prev
    _edge_math(ea, gs, gd, w1a, w1b, w1c, b1, w2, b2, out)


def _node_body(nr, p0, p1, w1a, w1b, b1, w2, b2, out):
    e2n = p0[...][0] + p1[...][0]
    g = (jnp.dot(nr[...], w1a[...], preferred_element_type=jnp.float32)
         + jnp.dot(e2n, w1b[...], preferred_element_type=jnp.float32)
         + b1[...])
    g = jnp.maximum(g, 0.0)
    out[...] = jnp.dot(g, w2[...], preferred_element_type=jnp.float32) + b2[...]


_BE = 2000
NBLK = E_SLICE // _BE
_BN = 1000


def _full(shape):
    return pl.BlockSpec(shape, lambda i: (0, 0))


def _edge_mlp_slice(q, eo_prev, edge_attr, gs, gd, weights):
    row_g = pl.BlockSpec((_BE, REP), lambda i, q=q: (q * NBLK + i, 0))
    row_l = pl.BlockSpec((_BE, REP), lambda i: (i, 0))
    wspecs = [_full((REP, HID)), _full((REP, HID)), _full((REP, HID)),
              _full((1, HID)), _full((HID, REP)), _full((1, REP))]
    if eo_prev is None:
        return pl.pallas_call(
            _edge_body_first,
            grid=(NBLK,),
            in_specs=[row_g, row_l, row_l] + wspecs,
            out_specs=row_g,
            out_shape=jax.ShapeDtypeStruct((N_EDGES, REP), jnp.float32),
        )(edge_attr, gs, gd, *weights)
    return pl.pallas_call(
        _edge_body_acc,
        grid=(NBLK,),
        in_specs=[pl.BlockSpec(memory_space=pl.ANY), row_g, row_l, row_l]
        + wspecs,
        out_specs=row_g,
        out_shape=jax.ShapeDtypeStruct((N_EDGES, REP), jnp.float32),
        input_output_aliases={0: 0},
    )(eo_prev, edge_attr, gs, gd, *weights)


def _node_mlp(node_rep, partials, w1a, w1b, b1, w2, b2):
    row = pl.BlockSpec((_BN, REP), lambda i: (i, 0))
    p0 = pl.BlockSpec((1, _BN, REP), lambda i: (0, i, 0))
    p1 = pl.BlockSpec((1, _BN, REP), lambda i: (1, i, 0))
    return pl.pallas_call(
        _node_body,
        grid=(N_NODES // _BN,),
        in_specs=[row, p0, p1,
                  _full((REP, HID)), _full((REP, HID)),
                  _full((1, HID)), _full((HID, REP)), _full((1, REP))],
        out_specs=row,
        out_shape=jax.ShapeDtypeStruct((N_NODES, REP), jnp.float32),
    )(node_rep, partials, partials, w1a, w1b, b1, w2, b2)


def kernel(node_rep, edge_index, edge_attr, We1, be1, We2, be2, Wn1, bn1, Wn2, bn2):
    bf = jnp.bfloat16
    src = edge_index[0].astype(jnp.int32)
    dst = edge_index[1].astype(jnp.int32)
    src_g = src.reshape(NSLICE, NC, NS, EPS)
    dst_g = dst.reshape(NSLICE, NC, NS, EPS)

    sc_gather = _build_sc_gather()
    gathered = [sc_gather(node_rep, src_g[q], dst_g[q]) for q in range(NSLICE)]

    weights = (We1[:REP].astype(bf), We1[REP:2 * REP].astype(bf),
               We1[2 * REP:].astype(bf), be1.reshape(1, HID),
               We2.astype(bf), be2.reshape(1, REP))
    edge_out = None
    for q in range(NSLICE):
        edge_out = _edge_mlp_slice(q, edge_out, edge_attr, *gathered[q],
                                   weights)

    zeros = jnp.zeros((N_NODES_PAD, REP), jnp.float32)
    eo_r = edge_out.reshape(NC, NS, S_NCHUNK, S_CHUNK, REP)
    src_sc = src.reshape(NC, NS, S_NCHUNK, S_CHUNK)
    dst_sc = dst.reshape(NC, NS, S_NCHUNK, S_CHUNK)
    partials = _build_sc_scatter()(eo_r, src_sc, dst_sc, zeros)

    node_out = _node_mlp(node_rep, partials, Wn1[:REP], Wn1[REP:],
                         bn1.reshape(1, HID), Wn2, bn2.reshape(1, REP))
    return node_out, edge_out


# speedup vs baseline: 1.2641x; 1.1471x over previous
import functools

import jax
import jax.numpy as jnp
from jax import lax
from jax.experimental import pallas as pl
from jax.experimental.pallas import tpu as pltpu
from jax.experimental.pallas import tpu_sc as plsc

REP = 128
HID = 2 * REP
N_NODES = 10000
N_EDGES = 320000

NC = 2
NS = 16
NW = NC * NS

NSLICE = 5
E_SLICE = N_EDGES // NSLICE
EPS = E_SLICE // NW
G_CHUNK = 80
G_NCHUNK = EPS // G_CHUNK
S_CHUNK = 80
EPW = N_EDGES // NW
S_NCHUNK = EPW // S_CHUNK

N_NODES_PAD = 10240
NPW = N_NODES_PAD // NS


@functools.cache
def _sc_mesh():
    return plsc.VectorSubcoreMesh(core_axis_name="c", subcore_axis_name="s")


@functools.cache
def _build_sc_gather():
    @functools.partial(
        pl.kernel,
        mesh=_sc_mesh(),
        out_type=(
            jax.ShapeDtypeStruct((E_SLICE, REP), jnp.float32),
            jax.ShapeDtypeStruct((E_SLICE, REP), jnp.float32),
        ),
        scratch_types=[
            pltpu.VMEM((G_NCHUNK, G_CHUNK), jnp.int32),
            pltpu.VMEM((G_NCHUNK, G_CHUNK), jnp.int32),
            pltpu.VMEM((2, G_CHUNK, REP), jnp.float32),
            pltpu.VMEM((2, G_CHUNK, REP), jnp.float32),
            pltpu.VMEM_SHARED((N_NODES_PAD, REP), jnp.float32),
            pltpu.SemaphoreType.DMA((2,)),
            pltpu.SemaphoreType.DMA((2,)),
            pltpu.SemaphoreType.DMA((2,)),
            pltpu.SemaphoreType.DMA((2,)),
        ],
    )
    def sc_gather(table, src_r, dst_r, gsrc, gdst,
                  idx_s, idx_d, rows_s, rows_d, tbl,
                  gsem_s, gsem_d, wsem_s, wsem_d):
        c = lax.axis_index("c")
        s = lax.axis_index("s")
        base = (c * NS + s) * EPS
        pltpu.sync_copy(table.at[pl.ds(s * NPW, NPW)], tbl.at[pl.ds(s * NPW, NPW)])
        pltpu.sync_copy(src_r.at[c, s], idx_s)
        pltpu.sync_copy(dst_r.at[c, s], idx_d)
        plsc.subcore_barrier()

        def start_gather(i, p):
            pltpu.async_copy(tbl.at[idx_s.at[i]], rows_s.at[p], gsem_s.at[p])
            pltpu.async_copy(tbl.at[idx_d.at[i]], rows_d.at[p], gsem_d.at[p])

        def wait_gather(i, p):
            pltpu.make_async_copy(tbl.at[idx_s.at[i]], rows_s.at[p],
                                  gsem_s.at[p]).wait()
            pltpu.make_async_copy(tbl.at[idx_d.at[i]], rows_d.at[p],
                                  gsem_d.at[p]).wait()

        def wait_write(i, p):
            off = base + i * G_CHUNK
            pltpu.make_async_copy(rows_s.at[p], gsrc.at[pl.ds(off, G_CHUNK)],
                                  wsem_s.at[p]).wait()
            pltpu.make_async_copy(rows_d.at[p], gdst.at[pl.ds(off, G_CHUNK)],
                                  wsem_d.at[p]).wait()

        start_gather(0, 0)

        def body(i, carry):
            p = i % 2

            @pl.when(i + 1 < G_NCHUNK)
            def _():
                @pl.when(i >= 1)
                def _():
                    wait_write(i - 1, 1 - p)
                start_gather(i + 1, 1 - p)

            wait_gather(i, p)
            off = base + i * G_CHUNK
            pltpu.async_copy(rows_s.at[p], gsrc.at[pl.ds(off, G_CHUNK)],
                             wsem_s.at[p])
            pltpu.async_copy(rows_d.at[p], gdst.at[pl.ds(off, G_CHUNK)],
                             wsem_d.at[p])
            return carry

        lax.fori_loop(0, G_NCHUNK, body, 0)
        wait_write(G_NCHUNK - 2, G_NCHUNK % 2)
        wait_write(G_NCHUNK - 1, (G_NCHUNK - 1) % 2)

    return sc_gather


@functools.cache
def _build_sc_scatter():
    @functools.partial(
        pl.kernel,
        mesh=_sc_mesh(),
        out_type=jax.ShapeDtypeStruct((NC, N_NODES_PAD, REP), jnp.float32),
        scratch_types=[
            pltpu.VMEM((2, S_CHUNK), jnp.int32),
            pltpu.VMEM((2, S_CHUNK), jnp.int32),
            pltpu.VMEM((2, S_CHUNK, REP), jnp.float32),
            pltpu.VMEM_SHARED((N_NODES_PAD, REP), jnp.float32),
            pltpu.SemaphoreType.DMA((2,)),
            pltpu.SemaphoreType.DMA((2,)),
            pltpu.SemaphoreType.DMA((2,)),
            pltpu.SemaphoreType.DMA((2,)),
            pltpu.SemaphoreType.DMA((2,)),
        ],
    )
    def sc_scatter(eo_r, src_r, dst_r, zeros, out, idx_s, idx_d, rows, acc,
                   rsem, isem_s, isem_d, asem_s, asem_d):
        c = lax.axis_index("c")
        s = lax.axis_index("s")
        pltpu.sync_copy(zeros.at[pl.ds(s * NPW, NPW)], acc.at[pl.ds(s * NPW, NPW)])
        plsc.subcore_barrier()

        def start_chunk(i, p):
            pltpu.async_copy(src_r.at[c, s, i], idx_s.at[p], isem_s.at[p])
            pltpu.async_copy(dst_r.at[c, s, i], idx_d.at[p], isem_d.at[p])
            pltpu.async_copy(eo_r.at[c, s, i], rows.at[p], rsem.at[p])

        def wait_chunk(i, p):
            pltpu.make_async_copy(src_r.at[c, s, i], idx_s.at[p],
                                  isem_s.at[p]).wait()
            pltpu.make_async_copy(dst_r.at[c, s, i], idx_d.at[p],
                                  isem_d.at[p]).wait()
            pltpu.make_async_copy(eo_r.at[c, s, i], rows.at[p],
                                  rsem.at[p]).wait()

        def start_adds(p):
            pltpu.async_copy(rows.at[p], acc.at[idx_s.at[p]], asem_s.at[p],
                             add=True)
            pltpu.async_copy(rows.at[p], acc.at[idx_d.at[p]], asem_d.at[p],
                             add=True)

        def wait_adds(p):
            pltpu.make_async_copy(rows.at[p], acc.at[idx_s.at[p]],
                                  asem_s.at[p]).wait()
            pltpu.make_async_copy(rows.at[p], acc.at[idx_d.at[p]],
                                  asem_d.at[p]).wait()

        start_chunk(0, 0)

        def body(i, carry):
            p = i % 2

            @pl.when(i + 1 < S_NCHUNK)
            def _():
                @pl.when(i >= 1)
                def _():
                    wait_adds(1 - p)
                start_chunk(i + 1, 1 - p)

            wait_chunk(i, p)
            start_adds(p)
            return carry

        lax.fori_loop(0, S_NCHUNK, body, 0)
        wait_adds(S_NCHUNK % 2)
        wait_adds((S_NCHUNK - 1) % 2)
        plsc.subcore_barrier()
        pltpu.sync_copy(acc.at[pl.ds(s * NPW, NPW)],
                        out.at[c].at[pl.ds(s * NPW, NPW)])

    return sc_scatter


def _edge_math(ea, gs, gd, w1a, w1b, w1c, b1, w2, b2, out):
    f32 = jnp.float32
    bf = jnp.bfloat16
    h = (jnp.dot(ea[...].astype(bf), w1a[...], preferred_element_type=f32)
         + jnp.dot(gs[...].astype(bf), w1b[...], preferred_element_type=f32)
         + jnp.dot(gd[...].astype(bf), w1c[...], preferred_element_type=f32)
         + b1[...])
    h = jnp.maximum(h, 0.0).astype(bf)
    out[...] = jnp.dot(h, w2[...], preferred_element_type=f32) + b2[...]


def _edge_body_first(ea, gs, gd, w1a, w1b, w1c, b1, w2, b2, out):
    _edge_math(ea, gs, gd, w1a, w1b, w1c, b1, w2, b2, out)


def _edge_body_acc(prev, ea, gs, gd, w1a, w1b, w1c, b1, w2, b2, out):
    del prev
    _edge_math(ea, gs, gd, w1a, w1b, w1c, b1, w2, b2, out)


def _node_body(nr, p0, p1, w1a, w1b, b1, w2, b2, out):
    e2n = p0[...][0] + p1[...][0]
    g = (jnp.dot(nr[...], w1a[...], preferred_element_type=jnp.float32)
         + jnp.dot(e2n, w1b[...], preferred_element_type=jnp.float32)
         + b1[...])
    g = jnp.maximum(g, 0.0)
    out[...] = jnp.dot(g, w2[...], preferred_element_type=jnp.float32) + b2[...]


_BE = 2000
NBLK = E_SLICE // _BE
_BN = 1000


def _full(shape):
    return pl.BlockSpec(shape, lambda i: (0, 0))


def _edge_mlp_slice(q, eo_prev, edge_attr, gs, gd, weights):
    row_g = pl.BlockSpec((_BE, REP), lambda i, q=q: (q * NBLK + i, 0))
    row_l = pl.BlockSpec((_BE, REP), lambda i: (i, 0))
    wspecs = [_full((REP, HID)), _full((REP, HID)), _full((REP, HID)),
              _full((1, HID)), _full((HID, REP)), _full((1, REP))]
    if eo_prev is None:
        return pl.pallas_call(
            _edge_body_first,
            grid=(NBLK,),
            in_specs=[row_g, row_l, row_l] + wspecs,
            out_specs=row_g,
            out_shape=jax.ShapeDtypeStruct((N_EDGES, REP), jnp.float32),
        )(edge_attr, gs, gd, *weights)
    return pl.pallas_call(
        _edge_body_acc,
        grid=(NBLK,),
        in_specs=[pl.BlockSpec(memory_space=pl.ANY), row_g, row_l, row_l]
        + wspecs,
        out_specs=row_g,
        out_shape=jax.ShapeDtypeStruct((N_EDGES, REP), jnp.float32),
        input_output_aliases={0: 0},
    )(eo_prev, edge_attr, gs, gd, *weights)


def _node_mlp(node_rep, partials, w1a, w1b, b1, w2, b2):
    row = pl.BlockSpec((_BN, REP), lambda i: (i, 0))
    p0 = pl.BlockSpec((1, _BN, REP), lambda i: (0, i, 0))
    p1 = pl.BlockSpec((1, _BN, REP), lambda i: (1, i, 0))
    return pl.pallas_call(
        _node_body,
        grid=(N_NODES // _BN,),
        in_specs=[row, p0, p1,
                  _full((REP, HID)), _full((REP, HID)),
                  _full((1, HID)), _full((HID, REP)), _full((1, REP))],
        out_specs=row,
        out_shape=jax.ShapeDtypeStruct((N_NODES, REP), jnp.float32),
    )(node_rep, partials, partials, w1a, w1b, b1, w2, b2)


def kernel(node_rep, edge_index, edge_attr, We1, be1, We2, be2, Wn1, bn1, Wn2, bn2):
    bf = jnp.bfloat16
    src = edge_index[0].astype(jnp.int32)
    dst = edge_index[1].astype(jnp.int32)
    src_g = src.reshape(NSLICE, NC, NS, G_NCHUNK, G_CHUNK)
    dst_g = dst.reshape(NSLICE, NC, NS, G_NCHUNK, G_CHUNK)

    node_pad = jnp.zeros((N_NODES_PAD, REP), jnp.float32).at[:N_NODES].set(node_rep)
    sc_gather = _build_sc_gather()
    gathered = [sc_gather(node_pad, src_g[q], dst_g[q]) for q in range(NSLICE)]

    weights = (We1[:REP].astype(bf), We1[REP:2 * REP].astype(bf),
               We1[2 * REP:].astype(bf), be1.reshape(1, HID),
               We2.astype(bf), be2.reshape(1, REP))
    edge_out = None
    for q in range(NSLICE):
        edge_out = _edge_mlp_slice(q, edge_out, edge_attr, *gathered[q],
                                   weights)

    zeros = jnp.zeros((N_NODES_PAD, REP), jnp.float32)
    eo_r = edge_out.reshape(NC, NS, S_NCHUNK, S_CHUNK, REP)
    src_sc = src.reshape(NC, NS, S_NCHUNK, S_CHUNK)
    dst_sc = dst.reshape(NC, NS, S_NCHUNK, S_CHUNK)
    partials = _build_sc_scatter()(eo_r, src_sc, dst_sc, zeros)

    node_out = _node_mlp(node_rep, partials, Wn1[:REP], Wn1[REP:],
                         bn1.reshape(1, HID), Wn2, bn2.reshape(1, REP))
    return node_out, edge_out
